# Initial kernel scaffold; baseline (speedup 1.0000x reference)
import jax
import jax.numpy as jnp
from jax.experimental import pallas as pl


def kernel(act_feat, src, W1, b1, W2, b2):
    raise NotImplementedError("write your pallas kernel here")


# speedup vs baseline: 7.9254x; 7.9254x over previous
import functools

import jax
import jax.numpy as jnp
from jax import lax
from jax.experimental import pallas as pl
from jax.experimental.pallas import tpu as pltpu
from jax.experimental.pallas import tpu_sc as plsc

N_EDGES = 320000
N_SEG = 10000
D = 128

NC = 2
NS = 16
L = 16
NW = NC * NS
EPT = N_EDGES // NW
CH = 80
NCHUNK = EPT // CH
SEG_PT = N_SEG // NS
ZCH = 125
NZ = SEG_PT // ZCH

BA = 2000
BC = 1000


def _score_body(x_ref, w1_ref, b1_ref, w2_ref, b2_ref, vals_ref, ex_ref):
    x = x_ref[...]
    h = jnp.tanh(jnp.dot(x, w1_ref[...], preferred_element_type=jnp.float32)
                 + b1_ref[...])
    s = jnp.dot(h, w2_ref[...], preferred_element_type=jnp.float32) + b2_ref[0, 0]
    c = jnp.sum(jnp.abs(w2_ref[...])) + jnp.abs(b2_ref[0, 0])
    e = jnp.exp(s - c)
    ex_ref[...] = e
    vals_ref[...] = x * e


_score_call = pl.pallas_call(
    _score_body,
    grid=(N_EDGES // BA,),
    in_specs=[
        pl.BlockSpec((BA, D), lambda i: (i, 0)),
        pl.BlockSpec((D, D), lambda i: (0, 0)),
        pl.BlockSpec((1, D), lambda i: (0, 0)),
        pl.BlockSpec((D, 1), lambda i: (0, 0)),
        pl.BlockSpec((1, 1), lambda i: (0, 0)),
    ],
    out_specs=[
        pl.BlockSpec((BA, D), lambda i: (i, 0)),
        pl.BlockSpec((BA, 1), lambda i: (i, 0)),
    ],
    out_shape=[
        jax.ShapeDtypeStruct((N_EDGES, D), jnp.float32),
        jax.ShapeDtypeStruct((N_EDGES, 1), jnp.float32),
    ],
)


_sc_mesh = plsc.VectorSubcoreMesh(
    core_axis_name="c", subcore_axis_name="s", num_cores=NC, num_subcores=NS)


@functools.partial(
    pl.kernel,
    out_type=(
        jax.ShapeDtypeStruct((NC, N_SEG, D), jnp.float32),
        jax.ShapeDtypeStruct((NC, N_SEG), jnp.float32),
        jax.ShapeDtypeStruct((NC, N_SEG), jnp.float32),
    ),
    mesh=_sc_mesh,
    scratch_types=[
        pltpu.VMEM_SHARED((N_SEG, D), jnp.float32),
        pltpu.VMEM_SHARED((N_SEG,), jnp.float32),
        pltpu.VMEM_SHARED((N_SEG,), jnp.float32),
        pltpu.VMEM((CH, D), jnp.float32),
        pltpu.VMEM((CH,), jnp.float32),
        pltpu.VMEM((CH,), jnp.int32),
        pltpu.VMEM((CH,), jnp.float32),
        pltpu.VMEM((ZCH, D), jnp.float32),
        pltpu.VMEM((CH,), jnp.float32),
    ],
)
def _seg_kernel(vals_hbm, ex_hbm, src_hbm, s_out, d_out, c_out,
                S_acc, d_acc, c_acc, row_buf, ex_buf, idx_buf,
                ones_buf, zrow_buf, zvec_buf):
    cid = lax.axis_index("c")
    sid = lax.axis_index("s")
    wid = sid * NC + cid

    def fill_row(i, _):
        def fill_lane(j, _):
            zrow_buf[i, pl.ds(j * L, L)] = jnp.zeros((L,), jnp.float32)
            return 0
        return lax.fori_loop(0, D // L, fill_lane, 0)
    lax.fori_loop(0, ZCH, fill_row, 0)

    def fill_vec(i, _):
        ones_buf[pl.ds(i * L, L)] = jnp.ones((L,), jnp.float32)
        zvec_buf[pl.ds(i * L, L)] = jnp.zeros((L,), jnp.float32)
        return 0
    lax.fori_loop(0, CH // L, fill_vec, 0)

    def zero_rows(i, _):
        pltpu.sync_copy(zrow_buf, S_acc.at[pl.ds(sid * SEG_PT + i * ZCH, ZCH)])
        return 0
    lax.fori_loop(0, NZ, zero_rows, 0)

    def zero_d(i, _):
        pltpu.sync_copy(zvec_buf, d_acc.at[pl.ds(i * CH, CH)])
        return 0

    def zero_c(i, _):
        pltpu.sync_copy(zvec_buf, c_acc.at[pl.ds(i * CH, CH)])
        return 0

    @pl.when(sid == 0)
    def _():
        lax.fori_loop(0, N_SEG // CH, zero_d, 0)

    @pl.when(sid == 1)
    def _():
        lax.fori_loop(0, N_SEG // CH, zero_c, 0)

    plsc.subcore_barrier()

    base0 = wid * EPT

    def chunk(i, _):
        b = base0 + i * CH
        pltpu.sync_copy(vals_hbm.at[pl.ds(b, CH)], row_buf)
        pltpu.sync_copy(ex_hbm.at[pl.ds(b, CH)], ex_buf)
        pltpu.sync_copy(src_hbm.at[pl.ds(b, CH)], idx_buf)
        pltpu.sync_copy(row_buf, S_acc.at[idx_buf], add=True)
        pltpu.sync_copy(ex_buf, d_acc.at[idx_buf], add=True)
        pltpu.sync_copy(ones_buf, c_acc.at[idx_buf], add=True)
        return 0
    lax.fori_loop(0, NCHUNK, chunk, 0)

    plsc.subcore_barrier()

    @pl.when(sid < 10)
    def _():
        r0 = sid * 1000
        pltpu.sync_copy(S_acc.at[pl.ds(r0, 1000)],
                        s_out.at[cid, pl.ds(r0, 1000)])

    @pl.when(sid == 0)
    def _():
        pltpu.sync_copy(d_acc, d_out.at[cid])
        pltpu.sync_copy(c_acc, c_out.at[cid])


def _final_body(s_ref, d_ref, c_ref, o_ref):
    s = s_ref[0] + s_ref[1]
    d = d_ref[0] + d_ref[1]
    c = c_ref[0] + c_ref[1]
    denom = d * jnp.maximum(c, 1.0)
    o_ref[...] = jnp.where(denom > 0.0, s / denom, 0.0)


_final_call = pl.pallas_call(
    _final_body,
    grid=(N_SEG // BC,),
    in_specs=[
        pl.BlockSpec((NC, BC, D), lambda i: (0, i, 0)),
        pl.BlockSpec((NC, BC, 1), lambda i: (0, i, 0)),
        pl.BlockSpec((NC, BC, 1), lambda i: (0, i, 0)),
    ],
    out_specs=pl.BlockSpec((BC, D), lambda i: (i, 0)),
    out_shape=jax.ShapeDtypeStruct((N_SEG, D), jnp.float32),
)


def kernel(act_feat, src, W1, b1, W2, b2):
    vals, ex = _score_call(act_feat, W1, b1.reshape(1, D), W2, b2.reshape(1, 1))
    s2, d2, c2 = _seg_kernel(vals, ex.reshape(N_EDGES), src.astype(jnp.int32))
    return _final_call(s2, d2.reshape(NC, N_SEG, 1), c2.reshape(NC, N_SEG, 1))


# speedup vs baseline: 11.5500x; 1.4573x over previous
import functools

import jax
import jax.numpy as jnp
from jax import lax
from jax.experimental import pallas as pl
from jax.experimental.pallas import tpu as pltpu
from jax.experimental.pallas import tpu_sc as plsc

N_EDGES = 320000
N_SEG = 10000
D = 128

NC = 2
NS = 16
L = 16
NW = NC * NS
EPT = N_EDGES // NW
CH = 80
NCHUNK = EPT // CH
SEG_PT = N_SEG // NS
ZCH = 125
NZ = SEG_PT // ZCH

BA = 2000
BC = 1000


def _score_body(x_ref, w1_ref, b1_ref, w2_ref, b2_ref, vals_ref, ex_ref):
    x = x_ref[...]
    h = jnp.tanh(jnp.dot(x, w1_ref[...], preferred_element_type=jnp.float32)
                 + b1_ref[...])
    s = jnp.dot(h, w2_ref[...], preferred_element_type=jnp.float32) + b2_ref[0, 0]
    c = jnp.sum(jnp.abs(w2_ref[...])) + jnp.abs(b2_ref[0, 0])
    e = jnp.exp(s - c)
    ex_ref[...] = e
    vals_ref[...] = x * e


_score_call = pl.pallas_call(
    _score_body,
    grid=(N_EDGES // BA,),
    in_specs=[
        pl.BlockSpec((BA, D), lambda i: (i, 0)),
        pl.BlockSpec((D, D), lambda i: (0, 0)),
        pl.BlockSpec((1, D), lambda i: (0, 0)),
        pl.BlockSpec((D, 1), lambda i: (0, 0)),
        pl.BlockSpec((1, 1), lambda i: (0, 0)),
    ],
    out_specs=[
        pl.BlockSpec((BA, D), lambda i: (i, 0)),
        pl.BlockSpec((BA, 1), lambda i: (i, 0)),
    ],
    out_shape=[
        jax.ShapeDtypeStruct((N_EDGES, D), jnp.float32),
        jax.ShapeDtypeStruct((N_EDGES, 1), jnp.float32),
    ],
)


_sc_mesh = plsc.VectorSubcoreMesh(
    core_axis_name="c", subcore_axis_name="s", num_cores=NC, num_subcores=NS)


@functools.partial(
    pl.kernel,
    out_type=(
        jax.ShapeDtypeStruct((NC, N_SEG, D), jnp.float32),
        jax.ShapeDtypeStruct((NC, N_SEG), jnp.float32),
        jax.ShapeDtypeStruct((NC, N_SEG), jnp.float32),
    ),
    mesh=_sc_mesh,
    scratch_types=[
        pltpu.VMEM_SHARED((N_SEG, D), jnp.float32),
        pltpu.VMEM_SHARED((N_SEG,), jnp.float32),
        pltpu.VMEM_SHARED((N_SEG,), jnp.float32),
        pltpu.VMEM((CH, D), jnp.float32),
        pltpu.VMEM((CH, D), jnp.float32),
        pltpu.VMEM((CH,), jnp.float32),
        pltpu.VMEM((CH,), jnp.float32),
        pltpu.VMEM((CH,), jnp.int32),
        pltpu.VMEM((CH,), jnp.int32),
        pltpu.VMEM((CH,), jnp.float32),
        pltpu.VMEM((ZCH, D), jnp.float32),
        pltpu.VMEM((2000,), jnp.float32),
        pltpu.SemaphoreType.DMA,
        pltpu.SemaphoreType.DMA,
    ],
)
def _seg_kernel(vals_hbm, ex_hbm, src_hbm, s_out, d_out, c_out,
                S_acc, d_acc, c_acc, row_a, row_b, ex_a, ex_b, idx_a, idx_b,
                ones_buf, zrow_buf, zvec_buf, sem_a, sem_b):
    cid = lax.axis_index("c")
    sid = lax.axis_index("s")
    wid = sid * NC + cid
    rows = (row_a, row_b)
    exs = (ex_a, ex_b)
    idxs = (idx_a, idx_b)
    sems = (sem_a, sem_b)

    def fill_row(i, _):
        def fill_lane(j, _):
            zrow_buf[i, pl.ds(j * L, L)] = jnp.zeros((L,), jnp.float32)
            return 0
        return lax.fori_loop(0, D // L, fill_lane, 0)
    lax.fori_loop(0, ZCH, fill_row, 0)

    def fill_vec(i, _):
        ones_buf[pl.ds(i * L, L)] = jnp.ones((L,), jnp.float32)
        return 0
    lax.fori_loop(0, CH // L, fill_vec, 0)

    def fill_z(i, _):
        zvec_buf[pl.ds(i * L, L)] = jnp.zeros((L,), jnp.float32)
        return 0
    lax.fori_loop(0, 2000 // L, fill_z, 0)

    def zero_rows(i, _):
        pltpu.sync_copy(zrow_buf, S_acc.at[pl.ds(sid * SEG_PT + i * ZCH, ZCH)])
        return 0
    lax.fori_loop(0, NZ, zero_rows, 0)

    @pl.when(sid < 5)
    def _():
        pltpu.sync_copy(zvec_buf, d_acc.at[pl.ds(sid * 2000, 2000)])

    @pl.when((sid >= 5) & (sid < 10))
    def _():
        pltpu.sync_copy(zvec_buf, c_acc.at[pl.ds((sid - 5) * 2000, 2000)])

    plsc.subcore_barrier()

    base0 = wid * EPT

    def start_loads(j, p):
        b = base0 + j * CH
        pltpu.async_copy(vals_hbm.at[pl.ds(b, CH)], rows[p], sems[p])
        pltpu.async_copy(ex_hbm.at[pl.ds(b, CH)], exs[p], sems[p])
        pltpu.async_copy(src_hbm.at[pl.ds(b, CH)], idxs[p], sems[p])

    def wait_loads(j, p):
        b = base0 + j * CH
        pltpu.make_async_copy(vals_hbm.at[pl.ds(b, CH)], rows[p], sems[p]).wait()
        pltpu.make_async_copy(ex_hbm.at[pl.ds(b, CH)], exs[p], sems[p]).wait()
        pltpu.make_async_copy(src_hbm.at[pl.ds(b, CH)], idxs[p], sems[p]).wait()

    def scatter(p):
        pltpu.sync_copy(rows[p], S_acc.at[idxs[p]], add=True)
        pltpu.sync_copy(exs[p], d_acc.at[idxs[p]], add=True)
        pltpu.sync_copy(ones_buf, c_acc.at[idxs[p]], add=True)

    start_loads(0, 0)

    def pair(i, _):
        j = 2 * i
        start_loads(j + 1, 1)
        wait_loads(j, 0)
        scatter(0)
        start_loads(j + 2, 0)
        wait_loads(j + 1, 1)
        scatter(1)
        return 0
    lax.fori_loop(0, (NCHUNK - 1) // 2, pair, 0)

    wait_loads(NCHUNK - 1, 0)
    scatter(0)

    plsc.subcore_barrier()

    @pl.when(sid < 10)
    def _():
        r0 = sid * 1000
        pltpu.sync_copy(S_acc.at[pl.ds(r0, 1000)],
                        s_out.at[cid, pl.ds(r0, 1000)])

    @pl.when(sid == 0)
    def _():
        pltpu.sync_copy(d_acc, d_out.at[cid])
        pltpu.sync_copy(c_acc, c_out.at[cid])


def _final_body(s_ref, d_ref, c_ref, o_ref):
    s = s_ref[0] + s_ref[1]
    d = d_ref[0] + d_ref[1]
    c = c_ref[0] + c_ref[1]
    denom = d * jnp.maximum(c, 1.0)
    o_ref[...] = jnp.where(denom > 0.0, s / denom, 0.0)


_final_call = pl.pallas_call(
    _final_body,
    grid=(N_SEG // BC,),
    in_specs=[
        pl.BlockSpec((NC, BC, D), lambda i: (0, i, 0)),
        pl.BlockSpec((NC, BC, 1), lambda i: (0, i, 0)),
        pl.BlockSpec((NC, BC, 1), lambda i: (0, i, 0)),
    ],
    out_specs=pl.BlockSpec((BC, D), lambda i: (i, 0)),
    out_shape=jax.ShapeDtypeStruct((N_SEG, D), jnp.float32),
)


def kernel(act_feat, src, W1, b1, W2, b2):
    vals, ex = _score_call(act_feat, W1, b1.reshape(1, D), W2, b2.reshape(1, 1))
    s2, d2, c2 = _seg_kernel(vals, ex.reshape(N_EDGES), src.astype(jnp.int32))
    return _final_call(s2, d2.reshape(NC, N_SEG, 1), c2.reshape(NC, N_SEG, 1))
